# baseline (speedup 1.0000x reference)
import jax
import jax.numpy as jnp
from jax.experimental import pallas as pl
from jax.experimental.pallas import tpu as pltpu

V = 1000
N = 3
CB = 64
NSEM = 8


def _fill_kernel(alpha_ref, batch_ref, out_hbm, tgt_ref, scratch, sems):
    a0 = alpha_ref[0, 0]
    a1 = alpha_ref[0, 1]
    a2 = alpha_ref[0, 2]
    s_full = (a0 + a1 + a2) * (1.0 / V)
    s_last = (a0 + a1) * (1.0 / V)
    n_preds = scratch.shape[1]

    tgt_ref[...] = batch_ref[:, N - 1:]

    scratch[:, : n_preds - 1, :] = jnp.full(
        (CB, n_preds - 1, V), 0.0, jnp.float32) + s_full
    scratch[:, n_preds - 1:, :] = jnp.full(
        (CB, 1, V), 0.0, jnp.float32) + s_last

    num_chunks = out_hbm.shape[0] // CB

    def start(i):
        pltpu.make_async_copy(
            scratch, out_hbm.at[pl.ds(i * CB, CB)], sems.at[i % NSEM]
        ).start()

    def wait(i):
        pltpu.make_async_copy(
            scratch, out_hbm.at[pl.ds(i * CB, CB)], sems.at[i % NSEM]
        ).wait()

    def body(i, carry):
        start(i)

        @pl.when(i >= NSEM)
        def _():
            wait(i - NSEM)

        return carry

    jax.lax.fori_loop(0, num_chunks, body, 0)

    def drain(i, carry):
        wait(num_chunks - NSEM + i)
        return carry

    jax.lax.fori_loop(0, min(NSEM, num_chunks), drain, 0)


def kernel(batch, TEXT, alpha):
    B, bptt = batch.shape
    n_preds = bptt - (N - 1) + 1
    n_tgt = n_preds - 1
    alpha2d = alpha.reshape(1, 3)

    outputs, targets = pl.pallas_call(
        _fill_kernel,
        in_specs=[
            pl.BlockSpec(memory_space=pltpu.MemorySpace.VMEM),
            pl.BlockSpec(memory_space=pltpu.MemorySpace.VMEM),
        ],
        out_specs=[
            pl.BlockSpec(memory_space=pltpu.MemorySpace.HBM),
            pl.BlockSpec(memory_space=pltpu.MemorySpace.VMEM),
        ],
        out_shape=[
            jax.ShapeDtypeStruct((B, n_preds, V), jnp.float32),
            jax.ShapeDtypeStruct((B, n_tgt), batch.dtype),
        ],
        scratch_shapes=[
            pltpu.VMEM((CB, n_preds, V), jnp.float32),
            pltpu.SemaphoreType.DMA((NSEM,)),
        ],
    )(alpha2d, batch)
    return outputs, targets


# speedup vs baseline: 1.0008x; 1.0008x over previous
import jax
import jax.numpy as jnp
from jax.experimental import pallas as pl
from jax.experimental.pallas import tpu as pltpu

V = 1000
N = 3
CB = 64
NSEM = 16


def _fill_kernel(alpha_ref, batch_ref, out_hbm, tgt_ref, scratch, sems):
    a0 = alpha_ref[0, 0]
    a1 = alpha_ref[0, 1]
    a2 = alpha_ref[0, 2]
    s_full = (a0 + a1 + a2) * (1.0 / V)
    s_last = (a0 + a1) * (1.0 / V)
    n_preds = scratch.shape[1]

    tgt_ref[...] = batch_ref[:, N - 1:]

    scratch[:, : n_preds - 1, :] = jnp.full(
        (CB, n_preds - 1, V), 0.0, jnp.float32) + s_full
    scratch[:, n_preds - 1:, :] = jnp.full(
        (CB, 1, V), 0.0, jnp.float32) + s_last

    num_chunks = out_hbm.shape[0] // CB

    for i in range(num_chunks):
        pltpu.make_async_copy(
            scratch, out_hbm.at[pl.ds(i * CB, CB)], sems.at[i % NSEM]
        ).start()
    for i in range(num_chunks):
        pltpu.make_async_copy(
            scratch, out_hbm.at[pl.ds(i * CB, CB)], sems.at[i % NSEM]
        ).wait()


def kernel(batch, TEXT, alpha):
    B, bptt = batch.shape
    n_preds = bptt - (N - 1) + 1
    n_tgt = n_preds - 1
    alpha2d = alpha.reshape(1, 3)

    outputs, targets = pl.pallas_call(
        _fill_kernel,
        in_specs=[
            pl.BlockSpec(memory_space=pltpu.MemorySpace.VMEM),
            pl.BlockSpec(memory_space=pltpu.MemorySpace.VMEM),
        ],
        out_specs=[
            pl.BlockSpec(memory_space=pltpu.MemorySpace.HBM),
            pl.BlockSpec(memory_space=pltpu.MemorySpace.VMEM),
        ],
        out_shape=[
            jax.ShapeDtypeStruct((B, n_preds, V), jnp.float32),
            jax.ShapeDtypeStruct((B, n_tgt), batch.dtype),
        ],
        scratch_shapes=[
            pltpu.VMEM((CB, n_preds, V), jnp.float32),
            pltpu.SemaphoreType.DMA((NSEM,)),
        ],
    )(alpha2d, batch)
    return outputs, targets
